# Initial kernel scaffold; baseline (speedup 1.0000x reference)
import jax
import jax.numpy as jnp
from jax.experimental import pallas as pl


def kernel(x, low, high):
    raise NotImplementedError("write your pallas kernel here")


# speedup vs baseline: 39.8643x; 39.8643x over previous
import functools

import numpy as np

import jax
import jax.numpy as jnp
from jax import lax
from jax.experimental import pallas as pl
from jax.experimental.pallas import tpu as pltpu
from jax.experimental.pallas import tpu_sc as plsc

DECAY = 0.99
MAX_SCALE = 1.0
Q_LOW = 5.0
Q_HIGH = 95.0

NC = 2
NS = 16
NW = NC * NS
LANES = 16
HIST1 = 4096 * 16
HIST2 = 4 * 256 * 16
CHUNK = 16384


def _f32_index(q_pct: float, n: int):
    q = np.float32(q_pct) / np.float32(100.0)
    pos = q * (np.float32(n) - np.float32(1.0))
    lo = int(np.floor(pos))
    hi = int(np.ceil(pos))
    hw = float(np.float32(pos - np.floor(pos)))
    return lo, hi, hw


def _hist_pass1_body(nchunks, per_tile, x_hbm, out_hbm, buf0, buf1, hist,
                     sem0, sem1):
    wid = lax.axis_index("c") * NS + lax.axis_index("s")
    base = wid * per_tile
    lane32 = lax.iota(jnp.int32, 16) + 32768
    ones = jnp.ones((16,), jnp.int32)

    @pl.loop(0, HIST1 // 16)
    def _zero(i):
        hist[pl.ds(i * 16, 16)] = jnp.zeros((16,), jnp.int32)

    def process(buf):
        @pl.loop(0, CHUNK // 16, unroll=8)
        def _p(i):
            xv = buf[pl.ds(i * 16, 16)]
            bits = lax.bitcast_convert_type(xv, jnp.int32)
            skey = bits ^ ((bits >> 31) & 0x7FFFFFFF)
            idx = ((skey >> 20) << 4) + lane32
            plsc.addupdate_scatter(hist, [idx], ones)

    pltpu.async_copy(x_hbm.at[pl.ds(base, CHUNK)], buf0, sem0)

    @pl.loop(0, nchunks, step=2)
    def _outer(g):
        nxt = (g + 1) & (nchunks - 1)
        pltpu.async_copy(x_hbm.at[pl.ds(base + nxt * CHUNK, CHUNK)], buf1, sem1)
        pltpu.make_async_copy(x_hbm.at[pl.ds(base, CHUNK)], buf0, sem0).wait()
        process(buf0)
        nxt2 = (g + 2) & (nchunks - 1)
        pltpu.async_copy(x_hbm.at[pl.ds(base + nxt2 * CHUNK, CHUNK)], buf0, sem0)
        pltpu.make_async_copy(x_hbm.at[pl.ds(base, CHUNK)], buf1, sem1).wait()
        process(buf1)

    pltpu.make_async_copy(x_hbm.at[pl.ds(base, CHUNK)], buf0, sem0).wait()
    pltpu.sync_copy(hist, out_hbm.at[wid])


def _hist_pass2_body(nchunks, per_tile, x_hbm, sel_hbm, out_hbm, buf0, buf1,
                     hist, pfxv, sem0, sem1):
    wid = lax.axis_index("c") * NS + lax.axis_index("s")
    base = wid * per_tile
    lane = lax.iota(jnp.int32, 16)
    ones = jnp.ones((16,), jnp.int32)

    pltpu.sync_copy(sel_hbm, pfxv)
    pfx = [pfxv[pl.ds(t * 16, 16)] for t in range(4)]

    @pl.loop(0, HIST2 // 16)
    def _zero(i):
        hist[pl.ds(i * 16, 16)] = jnp.zeros((16,), jnp.int32)

    def process(buf):
        @pl.loop(0, CHUNK // 16, unroll=4)
        def _p(i):
            xv = buf[pl.ds(i * 16, 16)]
            bits = lax.bitcast_convert_type(xv, jnp.int32)
            skey = bits ^ ((bits >> 31) & 0x7FFFFFFF)
            hi12 = skey >> 20
            idx = (((skey >> 12) & 255) << 4) + lane
            plsc.addupdate_scatter(hist, [idx], ones, mask=hi12 == pfx[0])
            plsc.addupdate_scatter(hist, [idx + 4096], ones, mask=hi12 == pfx[1])
            plsc.addupdate_scatter(hist, [idx + 8192], ones, mask=hi12 == pfx[2])
            plsc.addupdate_scatter(hist, [idx + 12288], ones, mask=hi12 == pfx[3])

    pltpu.async_copy(x_hbm.at[pl.ds(base, CHUNK)], buf0, sem0)

    @pl.loop(0, nchunks, step=2)
    def _outer(g):
        nxt = (g + 1) & (nchunks - 1)
        pltpu.async_copy(x_hbm.at[pl.ds(base + nxt * CHUNK, CHUNK)], buf1, sem1)
        pltpu.make_async_copy(x_hbm.at[pl.ds(base, CHUNK)], buf0, sem0).wait()
        process(buf0)
        nxt2 = (g + 2) & (nchunks - 1)
        pltpu.async_copy(x_hbm.at[pl.ds(base + nxt2 * CHUNK, CHUNK)], buf0, sem0)
        pltpu.make_async_copy(x_hbm.at[pl.ds(base, CHUNK)], buf1, sem1).wait()
        process(buf1)

    pltpu.make_async_copy(x_hbm.at[pl.ds(base, CHUNK)], buf0, sem0).wait()
    pltpu.sync_copy(hist, out_hbm.at[wid])


def _shifted(a, s, axis):
    rolled = pltpu.roll(a, s, axis)
    idx = lax.broadcasted_iota(jnp.int32, a.shape, axis)
    return jnp.where(idx >= s, rolled, 0)


def _flat_cumsum(h):
    rows, cols = h.shape
    a = h
    s = 1
    while s < cols:
        a = a + _shifted(a, s, 1)
        s *= 2
    rowtot = jnp.broadcast_to(a[:, cols - 1:cols], h.shape)
    b = rowtot
    s = 1
    while s < rows:
        b = b + _shifted(b, s, 0)
        s *= 2
    return a + b - rowtot


def _selb_body(ks, h_ref, out_ref, acc):
    i = pl.program_id(0)

    @pl.when(i == 0)
    def _():
        acc[...] = jnp.zeros_like(acc)

    acc[...] += h_ref[0]

    @pl.when(i == NW - 1)
    def _():
        h = acc[...]
        cum = _flat_cumsum(h)
        lanei = lax.broadcasted_iota(jnp.int32, h.shape, 1)
        rowi = lax.broadcasted_iota(jnp.int32, h.shape, 0)
        endm = (lanei & 15) == 15
        binidx = rowi * 8 + (lanei >> 4)
        for t, k in enumerate(ks):
            bstar = jnp.sum(jnp.where(endm & (cum <= k), 1, 0))
            cb = jnp.sum(jnp.where(binidx < bstar, h, 0))
            out_ref[0, t] = bstar - 2048
            out_ref[0, 4 + t] = k - cb
        out_ref[0, 8] = jnp.sum(h)
        for j in range(9, 16):
            out_ref[0, j] = 0


def _seld_body(hw5, hw95, h_ref, sel_ref, lh_ref, out_ref, acc):
    i = pl.program_id(0)

    @pl.when(i == 0)
    def _():
        acc[...] = jnp.zeros_like(acc)

    acc[...] += h_ref[0]

    @pl.when(i == NW - 1)
    def _():
        h = acc[...]
        cum = _flat_cumsum(h)
        lanei = lax.broadcasted_iota(jnp.int32, h.shape, 1)
        rowi = lax.broadcasted_iota(jnp.int32, h.shape, 0)
        endm = (lanei & 15) == 15
        vals = []
        for t in range(4):
            tmask = (rowi >= t * 32) & (rowi < (t + 1) * 32)
            base_t = jnp.sum(jnp.where(rowi < t * 32, h, 0))
            r_t = sel_ref[0, 4 + t]
            b2 = jnp.sum(jnp.where(endm & tmask & (cum <= base_t + r_t), 1, 0))
            skey = (sel_ref[0, t] << 20) | (b2 << 12) | 0x800
            bits = jnp.where(skey >= 0, skey, skey ^ 0x7FFFFFFF)
            vals.append(lax.bitcast_convert_type(bits, jnp.float32))
        p5 = (1.0 - hw5) * vals[0] + hw5 * vals[1]
        p95 = (1.0 - hw95) * vals[2] + hw95 * vals[3]
        new_low = DECAY * lh_ref[0, 0] + (1.0 - DECAY) * p5
        new_high = DECAY * lh_ref[0, 1] + (1.0 - DECAY) * p95
        out_ref[0, 0] = new_low
        out_ref[0, 1] = jnp.maximum(jnp.float32(1.0 / MAX_SCALE),
                                    new_high - new_low)


def kernel(x, low, high):
    n = x.size
    per_tile = n // NW
    nchunks = per_tile // CHUNK
    assert per_tile * NW == n and nchunks * CHUNK == per_tile
    assert nchunks & (nchunks - 1) == 0 and nchunks >= 2

    lo5, hi5, hw5 = _f32_index(Q_LOW, n)
    lo95, hi95, hw95 = _f32_index(Q_HIGH, n)
    ks = (lo5, hi5, lo95, hi95)

    xf = x.reshape(-1)
    mesh = plsc.VectorSubcoreMesh(core_axis_name="c", subcore_axis_name="s")
    sc_params = pltpu.CompilerParams(needs_layout_passes=False)

    hists1 = pl.kernel(
        functools.partial(_hist_pass1_body, nchunks, per_tile),
        out_type=jax.ShapeDtypeStruct((NW, HIST1), jnp.int32),
        mesh=mesh,
        compiler_params=sc_params,
        scratch_types=[
            pltpu.VMEM((CHUNK,), jnp.float32),
            pltpu.VMEM((CHUNK,), jnp.float32),
            pltpu.VMEM((HIST1,), jnp.int32),
            pltpu.SemaphoreType.DMA,
            pltpu.SemaphoreType.DMA,
        ],
    )(xf)

    selb = pl.pallas_call(
        functools.partial(_selb_body, ks),
        grid=(NW,),
        in_specs=[pl.BlockSpec((1, 512, 128), lambda i: (i, 0, 0))],
        out_specs=pl.BlockSpec(memory_space=pltpu.SMEM),
        out_shape=jax.ShapeDtypeStruct((1, 16), jnp.int32),
        scratch_shapes=[pltpu.VMEM((512, 128), jnp.int32)],
    )(hists1.reshape(NW, 512, 128))

    hists2 = pl.kernel(
        functools.partial(_hist_pass2_body, nchunks, per_tile),
        out_type=jax.ShapeDtypeStruct((NW, HIST2), jnp.int32),
        mesh=mesh,
        compiler_params=sc_params,
        scratch_types=[
            pltpu.VMEM((CHUNK,), jnp.float32),
            pltpu.VMEM((CHUNK,), jnp.float32),
            pltpu.VMEM((HIST2,), jnp.int32),
            pltpu.VMEM((64,), jnp.int32),
            pltpu.SemaphoreType.DMA,
            pltpu.SemaphoreType.DMA,
        ],
    )(xf, jnp.repeat(selb[0, :4], 16))

    lh = jnp.stack([low, high]).astype(jnp.float32).reshape(1, 2)
    res = pl.pallas_call(
        functools.partial(_seld_body, hw5, hw95),
        grid=(NW,),
        in_specs=[
            pl.BlockSpec((1, 128, 128), lambda i: (i, 0, 0)),
            pl.BlockSpec(memory_space=pltpu.SMEM),
            pl.BlockSpec(memory_space=pltpu.SMEM),
        ],
        out_specs=pl.BlockSpec(memory_space=pltpu.SMEM),
        out_shape=jax.ShapeDtypeStruct((1, 2), jnp.float32),
        scratch_shapes=[pltpu.VMEM((128, 128), jnp.int32)],
    )(hists2.reshape(NW, 128, 128), selb, lh)

    return (res[0, 0], res[0, 1])
